# Initial kernel scaffold; baseline (speedup 1.0000x reference)
import jax
import jax.numpy as jnp
from jax.experimental import pallas as pl


def kernel(x, edge_index, batch, edge_attr, W_gat, att_src, att_dst, bias_gat, pool_w, lin1_W, lin1_b, lin2_W, lin2_b, lin3_W, lin3_b):
    raise NotImplementedError("write your pallas kernel here")


# speedup vs baseline: 27.8111x; 27.8111x over previous
import functools
import jax
import jax.numpy as jnp
from jax import lax
from jax.experimental import pallas as pl
from jax.experimental.pallas import tpu as pltpu, tpu_sc as plsc

N = 10000
E = 160000
G = 64
H = 4
C = 128
NP = 10240
EP = 163840
RW = 48
UW = H * 32
NB = 1024
NBLK = NP // NB
CH = 64
NC = 2
NS = 16
EPW = EP // (NC * NS)
NCHUNK = EPW // CH
RPS = NP // NS


def _tca_body(x_ref, w_ref, ats_ref, atd_ref, o_ref):
    wr = w_ref[...].reshape(32, H, C)
    ats = ats_ref[...]
    atd = atd_ref[...]
    cols = []
    for h in range(H):
        cols.append(jnp.sum(wr[:, h, :] * ats[h:h + 1, :], axis=1,
                            keepdims=True))
    for h in range(H):
        cols.append(jnp.sum(wr[:, h, :] * atd[h:h + 1, :], axis=1,
                            keepdims=True))
    cols.append(jnp.zeros((32, 120), jnp.float32))
    v = jnp.concatenate(cols, axis=1)
    o_ref[...] = jnp.dot(x_ref[...], v, preferred_element_type=jnp.float32)


def _sc_body(x_hbm, src_hbm, dst_hbm, xd_hbm, outf_hbm, outd_hbm,
             srcv, dstv, rows_v, rowsd_v, out_v, exb_v, uf_sh, ud_sh, sem):
    cid = lax.axis_index("c")
    sid = lax.axis_index("s")
    zz = jnp.zeros((16,), jnp.float32)
    for e in range(CH):
        for t in range(UW // 16):
            out_v[e, pl.ds(t * 16, 16)] = zz
        exb_v[e, pl.ds(0, 16)] = zz
    for b in range(RPS // CH):
        pltpu.sync_copy(out_v, uf_sh.at[pl.ds(sid * RPS + b * CH, CH)])
        pltpu.sync_copy(exb_v, ud_sh.at[pl.ds(sid * RPS + b * CH, CH)])
    plsc.subcore_barrier()

    base0 = (cid * NS + sid) * EPW
    iota16 = lax.iota(jnp.int32, 16)

    def body(i, carry):
        base = base0 + i * CH
        pltpu.sync_copy(src_hbm.at[pl.ds(base, CH)], srcv)
        pltpu.sync_copy(dst_hbm.at[pl.ds(base, CH)], dstv)
        pltpu.async_copy(x_hbm.at[srcv], rows_v, sem).wait()
        pltpu.async_copy(xd_hbm.at[dstv], rowsd_v, sem).wait()
        for ee in range(CH):
            v = rows_v[ee, pl.ds(32, 16)]
            w = rowsd_v[ee, pl.ds(0, 16)]
            al = v + w
            al = jnp.maximum(al, al * 0.2)
            exv = jnp.exp(al)
            exb_v[ee, pl.ds(0, 16)] = exv
            r0 = rows_v[ee, pl.ds(0, 16)]
            r1 = rows_v[ee, pl.ds(16, 16)]
            for h in range(H):
                bc = exv[1 + h]
                out_v[ee, pl.ds(h * 32, 16)] = r0 * bc
                out_v[ee, pl.ds(h * 32 + 16, 16)] = r1 * bc
        pltpu.sync_copy(out_v, uf_sh.at[dstv], add=True)
        pltpu.sync_copy(exb_v, ud_sh.at[dstv], add=True)
        return carry

    lax.fori_loop(0, NCHUNK, body, 0)
    plsc.subcore_barrier()
    pltpu.sync_copy(uf_sh.at[pl.ds(sid * RPS, RPS)],
                    outf_hbm.at[cid, pl.ds(sid * RPS, RPS)])
    pltpu.sync_copy(ud_sh.at[pl.ds(sid * RPS, RPS)],
                    outd_hbm.at[cid, pl.ds(sid * RPS, RPS)])


def _sc_edge(x_aug, srcp, dstp, xd):
    mesh = plsc.VectorSubcoreMesh(core_axis_name="c", subcore_axis_name="s")
    k = functools.partial(
        pl.kernel,
        mesh=mesh,
        compiler_params=pltpu.CompilerParams(use_tc_tiling_on_sc=False),
        out_type=[
            jax.ShapeDtypeStruct((NC, NP, UW), jnp.float32),
            jax.ShapeDtypeStruct((NC, NP, 16), jnp.float32),
        ],
        scratch_types=[
            pltpu.VMEM((CH,), jnp.int32),
            pltpu.VMEM((CH,), jnp.int32),
            pltpu.VMEM((CH, RW), jnp.float32),
            pltpu.VMEM((CH, 16), jnp.float32),
            pltpu.VMEM((CH, UW), jnp.float32),
            pltpu.VMEM((CH, 16), jnp.float32),
            pltpu.VMEM_SHARED((NP, UW), jnp.float32),
            pltpu.VMEM_SHARED((NP, 16), jnp.float32),
            pltpu.SemaphoreType.DMA,
        ],
    )(_sc_body)
    return k(x_aug, srcp, dstp, xd)


def _tcb1_body(uf_ref, ud_ref, x_ref, a_ref, w_ref, b_ref, pw_ref,
               xw_ref, sc_ref):
    uf = uf_ref[0] + uf_ref[1]
    ud = ud_ref[0] + ud_ref[1]
    a = a_ref[...]
    asr = a[:, 0:H]
    adr = a[:, H:2 * H]
    als = asr + adr
    als = jnp.maximum(als, als * 0.2)
    exs = jnp.exp(als)
    xa = x_ref[...][:, :32]
    u4 = uf.reshape(NB, H, 32) + exs[:, :, None] * xa[:, None, :]
    den = jnp.maximum(ud[:, 1:1 + H] + exs, 1e-30)
    z = u4 / den[:, :, None]
    wr = w_ref[...].reshape(32, H, C)
    parts = [jnp.dot(z[:, h, :], wr[:, h, :], preferred_element_type=jnp.float32)
             for h in range(H)]
    agg = jnp.concatenate(parts, axis=1)
    x1 = jnp.maximum(agg + b_ref[...], 0.0)
    pw = pw_ref[...]
    nrm = jnp.sqrt(jnp.sum(pw * pw))
    sc = jnp.tanh(jnp.sum(x1 * pw, axis=1, keepdims=True) / nrm)
    sc_ref[...] = sc
    xw_ref[...] = x1 * sc


def _tcb1(u2f, u2d, x_aug, a, W_gat, bias2, pw2):
    return pl.pallas_call(
        _tcb1_body,
        grid=(NBLK,),
        in_specs=[
            pl.BlockSpec((NC, NB, UW), lambda i: (0, i, 0)),
            pl.BlockSpec((NC, NB, 16), lambda i: (0, i, 0)),
            pl.BlockSpec((NB, RW), lambda i: (i, 0)),
            pl.BlockSpec((NB, 128), lambda i: (i, 0)),
            pl.BlockSpec((32, H * C), lambda i: (0, 0)),
            pl.BlockSpec((1, H * C), lambda i: (0, 0)),
            pl.BlockSpec((1, H * C), lambda i: (0, 0)),
        ],
        out_specs=[
            pl.BlockSpec((NB, H * C), lambda i: (i, 0)),
            pl.BlockSpec((NB, 1), lambda i: (i, 0)),
        ],
        out_shape=[
            jax.ShapeDtypeStruct((NP, H * C), jnp.float32),
            jax.ShapeDtypeStruct((NP, 1), jnp.float32),
        ],
    )(u2f, u2d, x_aug, a, W_gat, bias2, pw2)


def _tcb2_body(xw_ref, scc_ref, scr_ref, batc_ref, batr_ref,
               l1_ref, b1_ref, l2_ref, b2_ref, l3_ref, b3_ref, o_ref,
               gmp_scr, gap_scr, cnt_scr, kofb_scr):
    i = pl.program_id(0)

    @pl.when(i == 0)
    def _init():
        gmp_scr[...] = jnp.full((G, H * C), -1e30, jnp.float32)
        gap_scr[...] = jnp.zeros((G, H * C), jnp.float32)
        cnt_scr[...] = jnp.zeros((G, 128), jnp.float32)
        bc = batc_ref[...]
        gids = lax.broadcasted_iota(jnp.int32, (NP, G), 1)
        mng = (bc == gids).astype(jnp.float32)
        counts = jnp.sum(mng, axis=0, keepdims=True)
        kk = jnp.ceil(0.8 * counts)
        kofb_scr[...] = jnp.sum(mng * kk, axis=1, keepdims=True)

    r0 = i * NB
    sc_r = scc_ref[pl.ds(r0, NB), :]
    bat_r = batc_ref[pl.ds(r0, NB), :]
    lj_lt_li = (lax.broadcasted_iota(jnp.int32, (NB, NB), 1)
                < lax.broadcasted_iota(jnp.int32, (NB, NB), 0))
    cnt = jnp.zeros((NB, 1), jnp.float32)
    for ct in range(NBLK):
        sc_c = scr_ref[:, pl.ds(ct * NB, NB)]
        bat_c = batr_ref[:, pl.ds(ct * NB, NB)]
        tie_ok = (ct < i) | ((ct == i) & lj_lt_li)
        prec = (sc_c > sc_r) | ((sc_c == sc_r) & tie_ok)
        same = bat_c == bat_r
        cnt = cnt + jnp.sum((prec & same).astype(jnp.float32), axis=1,
                            keepdims=True)
    sel = (cnt < kofb_scr[pl.ds(r0, NB), :]) & (bat_r < G)

    gids = lax.broadcasted_iota(jnp.int32, (NB, G), 1)
    m = ((bat_r == gids) & sel).astype(jnp.float32)
    xwb = xw_ref[...]
    gap_scr[...] += lax.dot_general(m, xwb, (((0,), (0,)), ((), ())),
                                    preferred_element_type=jnp.float32)
    cnt_scr[...] += lax.dot_general(m, jnp.ones((NB, 128), jnp.float32),
                                    (((0,), (0,)), ((), ())),
                                    preferred_element_type=jnp.float32)
    for g in range(G):
        colm = m[:, g:g + 1] > 0.0
        vals = jnp.where(colm, xwb, -1e30)
        mx = jnp.max(vals, axis=0, keepdims=True)
        gmp_scr[g:g + 1, :] = jnp.maximum(gmp_scr[g:g + 1, :], mx)

    @pl.when(i == NBLK - 1)
    def _final():
        selc = jnp.maximum(cnt_scr[:, 0:1], 1.0)
        x2 = jnp.concatenate([gmp_scr[...], gap_scr[...] / selc], axis=1)
        h1 = jnp.maximum(jnp.dot(x2, l1_ref[...],
                                 preferred_element_type=jnp.float32)
                         + b1_ref[...], 0.0)
        h2 = jnp.maximum(jnp.dot(h1, l2_ref[...],
                                 preferred_element_type=jnp.float32)
                         + b2_ref[...], 0.0)
        logits = jnp.dot(h2, l3_ref[...],
                         preferred_element_type=jnp.float32) + b3_ref[...]
        mx = jnp.max(logits, axis=1, keepdims=True)
        lse = jnp.log(jnp.sum(jnp.exp(logits - mx), axis=1, keepdims=True))
        o_ref[...] = logits - mx - lse


def _tcb2(xw, scc, scr, batc, batr, l1, b1, l2p, b2p, l3p, b3p):
    return pl.pallas_call(
        _tcb2_body,
        grid=(NBLK,),
        in_specs=[
            pl.BlockSpec((NB, H * C), lambda i: (i, 0)),
            pl.BlockSpec((NP, 1), lambda i: (0, 0)),
            pl.BlockSpec((1, NP), lambda i: (0, 0)),
            pl.BlockSpec((NP, 1), lambda i: (0, 0)),
            pl.BlockSpec((1, NP), lambda i: (0, 0)),
            pl.BlockSpec((2 * H * C, 128), lambda i: (0, 0)),
            pl.BlockSpec((1, 128), lambda i: (0, 0)),
            pl.BlockSpec((128, 128), lambda i: (0, 0)),
            pl.BlockSpec((1, 128), lambda i: (0, 0)),
            pl.BlockSpec((128, 128), lambda i: (0, 0)),
            pl.BlockSpec((1, 128), lambda i: (0, 0)),
        ],
        out_specs=pl.BlockSpec((G, 128), lambda i: (0, 0)),
        out_shape=jax.ShapeDtypeStruct((G, 128), jnp.float32),
        scratch_shapes=[
            pltpu.VMEM((G, H * C), jnp.float32),
            pltpu.VMEM((G, H * C), jnp.float32),
            pltpu.VMEM((G, 128), jnp.float32),
            pltpu.VMEM((NP, 1), jnp.float32),
        ],
    )(xw, scc, scr, batc, batr, l1, b1, l2p, b2p, l3p, b3p)


def kernel(x, edge_index, batch, edge_attr, W_gat, att_src, att_dst, bias_gat,
           pool_w, lin1_W, lin1_b, lin2_W, lin2_b, lin3_W, lin3_b):
    f32 = jnp.float32
    xp = jnp.concatenate([x, jnp.zeros((NP - N, 32), f32)], axis=0)
    ones = jnp.concatenate([jnp.ones((N, 1), f32), jnp.zeros((NP - N, 1), f32)],
                           axis=0)

    a = pl.pallas_call(
        _tca_body,
        out_shape=jax.ShapeDtypeStruct((NP, 128), jnp.float32),
    )(xp, W_gat, att_src, att_dst)

    x_aug = jnp.concatenate(
        [xp, ones, a[:, 0:H], jnp.zeros((NP, RW - 37), f32)], axis=1)
    xd = jnp.concatenate(
        [jnp.zeros((NP, 1), f32), a[:, H:2 * H], jnp.zeros((NP, 11), f32)],
        axis=1)
    pad_e = jnp.full((EP - E,), N, jnp.int32)
    srcp = jnp.concatenate([edge_index[0], pad_e])
    dstp = jnp.concatenate([edge_index[1], pad_e])

    u2f, u2d = _sc_edge(x_aug, srcp, dstp, xd)

    bias2 = bias_gat.reshape(1, H * C)
    pw2 = pool_w.reshape(1, H * C)
    xw, score = _tcb1(u2f, u2d, x_aug, a, W_gat, bias2, pw2)

    batp = jnp.concatenate([batch, jnp.full((NP - N,), G, jnp.int32)])
    batc = batp.reshape(NP, 1)
    batr = batp.reshape(1, NP)
    scc = score
    scr = score.reshape(1, NP)

    l2p = jnp.zeros((128, 128), f32).at[:, :64].set(lin2_W)
    b2p = jnp.zeros((1, 128), f32).at[0, :64].set(lin2_b)
    l3p = jnp.zeros((128, 128), f32).at[:64, :8].set(lin3_W)
    b3p = jnp.full((1, 128), -1e30, f32).at[0, :8].set(lin3_b)
    b1p = lin1_b.reshape(1, 128)

    out = _tcb2(xw, scc, scr, batc, batr, lin1_W, b1p, l2p, b2p, l3p, b3p)
    return out[:, :8]


# speedup vs baseline: 37.2783x; 1.3404x over previous
import functools
import jax
import jax.numpy as jnp
from jax import lax
from jax.experimental import pallas as pl
from jax.experimental.pallas import tpu as pltpu, tpu_sc as plsc

N = 10000
E = 160000
G = 64
H = 4
C = 128
NP = 10240
EP = 163840
RW = 48
UW = H * 32
NB = 1024
NBLK = NP // NB
CH = 64
NC = 2
NS = 16
EPW = EP // (NC * NS)
NCHUNK = EPW // CH
RPS = NP // NS


def _tca_body(x_ref, w_ref, ats_ref, atd_ref, o_ref):
    wr = w_ref[...].reshape(32, H, C)
    ats = ats_ref[...]
    atd = atd_ref[...]
    cols = []
    for h in range(H):
        cols.append(jnp.sum(wr[:, h, :] * ats[h:h + 1, :], axis=1,
                            keepdims=True))
    for h in range(H):
        cols.append(jnp.sum(wr[:, h, :] * atd[h:h + 1, :], axis=1,
                            keepdims=True))
    cols.append(jnp.zeros((32, 120), jnp.float32))
    v = jnp.concatenate(cols, axis=1)
    o_ref[...] = jnp.dot(x_ref[...], v, preferred_element_type=jnp.float32)


def _sc_body(x_hbm, src_hbm, dst_hbm, xd_hbm, outf_hbm, outd_hbm,
             srcall_v, dstall_v, rows0_v, rowsd0_v, rows1_v, rowsd1_v,
             out_v, exb_v, uf_sh, ud_sh, sem0, sem1):
    cid = lax.axis_index("c")
    sid = lax.axis_index("s")
    zz = jnp.zeros((16,), jnp.float32)
    for e in range(CH):
        for t in range(UW // 16):
            out_v[e, pl.ds(t * 16, 16)] = zz
        exb_v[e, pl.ds(0, 16)] = zz
    for b in range(RPS // CH):
        pltpu.sync_copy(out_v, uf_sh.at[pl.ds(sid * RPS + b * CH, CH)])
        pltpu.sync_copy(exb_v, ud_sh.at[pl.ds(sid * RPS + b * CH, CH)])

    wid = cid * NS + sid
    pltpu.sync_copy(src_hbm.at[pl.ds(wid * NCHUNK, NCHUNK + 1)], srcall_v)
    pltpu.sync_copy(dst_hbm.at[pl.ds(wid * NCHUNK, NCHUNK + 1)], dstall_v)
    plsc.subcore_barrier()

    def fetch(i, rows_b, rowsd_b, sem_b):
        pltpu.async_copy(x_hbm.at[srcall_v.at[i]], rows_b, sem_b)
        pltpu.async_copy(xd_hbm.at[dstall_v.at[i]], rowsd_b, sem_b)

    def process(i, rows_b, rowsd_b, sem_b):
        pltpu.make_async_copy(x_hbm.at[srcall_v.at[i]], rows_b, sem_b).wait()
        pltpu.make_async_copy(xd_hbm.at[dstall_v.at[i]], rowsd_b, sem_b).wait()
        for ee in range(CH):
            v = rows_b[ee, pl.ds(32, 16)]
            w = rowsd_b[ee, pl.ds(0, 16)]
            al = v + w
            al = jnp.maximum(al, al * 0.2)
            exv = jnp.exp(al)
            exb_v[ee, pl.ds(0, 16)] = exv
            r0 = rows_b[ee, pl.ds(0, 16)]
            r1 = rows_b[ee, pl.ds(16, 16)]
            for h in range(H):
                bc = exv[1 + h]
                out_v[ee, pl.ds(h * 32, 16)] = r0 * bc
                out_v[ee, pl.ds(h * 32 + 16, 16)] = r1 * bc
        pltpu.sync_copy(out_v, uf_sh.at[dstall_v.at[i]], add=True)
        pltpu.sync_copy(exb_v, ud_sh.at[dstall_v.at[i]], add=True)

    fetch(0, rows0_v, rowsd0_v, sem0)

    def body2(k, carry):
        i0 = 2 * k
        fetch(i0 + 1, rows1_v, rowsd1_v, sem1)
        process(i0, rows0_v, rowsd0_v, sem0)
        fetch(i0 + 2, rows0_v, rowsd0_v, sem0)
        process(i0 + 1, rows1_v, rowsd1_v, sem1)
        return carry

    lax.fori_loop(0, NCHUNK // 2, body2, 0)
    pltpu.make_async_copy(x_hbm.at[srcall_v.at[NCHUNK]], rows0_v, sem0).wait()
    pltpu.make_async_copy(xd_hbm.at[dstall_v.at[NCHUNK]], rowsd0_v,
                          sem0).wait()
    plsc.subcore_barrier()
    pltpu.sync_copy(uf_sh.at[pl.ds(sid * RPS, RPS)],
                    outf_hbm.at[cid, pl.ds(sid * RPS, RPS)])
    pltpu.sync_copy(ud_sh.at[pl.ds(sid * RPS, RPS)],
                    outd_hbm.at[cid, pl.ds(sid * RPS, RPS)])


def _sc_edge(x_aug, srcp, dstp, xd):
    mesh = plsc.VectorSubcoreMesh(core_axis_name="c", subcore_axis_name="s")
    k = functools.partial(
        pl.kernel,
        mesh=mesh,
        compiler_params=pltpu.CompilerParams(use_tc_tiling_on_sc=False),
        out_type=[
            jax.ShapeDtypeStruct((NC, NP, UW), jnp.float32),
            jax.ShapeDtypeStruct((NC, NP, 16), jnp.float32),
        ],
        scratch_types=[
            pltpu.VMEM((NCHUNK + 1, CH), jnp.int32),
            pltpu.VMEM((NCHUNK + 1, CH), jnp.int32),
            pltpu.VMEM((CH, RW), jnp.float32),
            pltpu.VMEM((CH, 16), jnp.float32),
            pltpu.VMEM((CH, RW), jnp.float32),
            pltpu.VMEM((CH, 16), jnp.float32),
            pltpu.VMEM((CH, UW), jnp.float32),
            pltpu.VMEM((CH, 16), jnp.float32),
            pltpu.VMEM_SHARED((NP, UW), jnp.float32),
            pltpu.VMEM_SHARED((NP, 16), jnp.float32),
            pltpu.SemaphoreType.DMA,
            pltpu.SemaphoreType.DMA,
        ],
    )(_sc_body)
    return k(x_aug, srcp, dstp, xd)


def _tcb1_body(uf_ref, ud_ref, x_ref, a_ref, w_ref, b_ref, pw_ref,
               xw_ref, sc_ref):
    uf = uf_ref[0] + uf_ref[1]
    ud = ud_ref[0] + ud_ref[1]
    a = a_ref[...]
    asr = a[:, 0:H]
    adr = a[:, H:2 * H]
    als = asr + adr
    als = jnp.maximum(als, als * 0.2)
    exs = jnp.exp(als)
    xa = x_ref[...][:, :32]
    u4 = uf.reshape(NB, H, 32) + exs[:, :, None] * xa[:, None, :]
    den = jnp.maximum(ud[:, 1:1 + H] + exs, 1e-30)
    z = u4 / den[:, :, None]
    wr = w_ref[...].reshape(32, H, C)
    parts = [jnp.dot(z[:, h, :], wr[:, h, :], preferred_element_type=jnp.float32)
             for h in range(H)]
    agg = jnp.concatenate(parts, axis=1)
    x1 = jnp.maximum(agg + b_ref[...], 0.0)
    pw = pw_ref[...]
    nrm = jnp.sqrt(jnp.sum(pw * pw))
    sc = jnp.tanh(jnp.sum(x1 * pw, axis=1, keepdims=True) / nrm)
    sc_ref[...] = sc
    xw_ref[...] = x1 * sc


def _tcb1(u2f, u2d, x_aug, a, W_gat, bias2, pw2):
    return pl.pallas_call(
        _tcb1_body,
        grid=(NBLK,),
        in_specs=[
            pl.BlockSpec((NC, NB, UW), lambda i: (0, i, 0)),
            pl.BlockSpec((NC, NB, 16), lambda i: (0, i, 0)),
            pl.BlockSpec((NB, RW), lambda i: (i, 0)),
            pl.BlockSpec((NB, 128), lambda i: (i, 0)),
            pl.BlockSpec((32, H * C), lambda i: (0, 0)),
            pl.BlockSpec((1, H * C), lambda i: (0, 0)),
            pl.BlockSpec((1, H * C), lambda i: (0, 0)),
        ],
        out_specs=[
            pl.BlockSpec((NB, H * C), lambda i: (i, 0)),
            pl.BlockSpec((NB, 1), lambda i: (i, 0)),
        ],
        out_shape=[
            jax.ShapeDtypeStruct((NP, H * C), jnp.float32),
            jax.ShapeDtypeStruct((NP, 1), jnp.float32),
        ],
    )(u2f, u2d, x_aug, a, W_gat, bias2, pw2)


def _tcb2_body(xw_ref, scc_ref, scr_ref, batc_ref, batr_ref,
               l1_ref, b1_ref, l2_ref, b2_ref, l3_ref, b3_ref, o_ref,
               gmp_scr, gap_scr, cnt_scr, kofb_scr):
    i = pl.program_id(0)

    @pl.when(i == 0)
    def _init():
        gmp_scr[...] = jnp.full((G, H * C), -1e30, jnp.float32)
        gap_scr[...] = jnp.zeros((G, H * C), jnp.float32)
        cnt_scr[...] = jnp.zeros((G, 128), jnp.float32)
        bc = batc_ref[...]
        gids = lax.broadcasted_iota(jnp.int32, (NP, G), 1)
        mng = (bc == gids).astype(jnp.float32)
        counts = jnp.sum(mng, axis=0, keepdims=True)
        kk = jnp.ceil(0.8 * counts)
        kofb_scr[...] = jnp.sum(mng * kk, axis=1, keepdims=True)

    r0 = i * NB
    sc_r = scc_ref[pl.ds(r0, NB), :]
    bat_r = batc_ref[pl.ds(r0, NB), :]
    lj_lt_li = (lax.broadcasted_iota(jnp.int32, (NB, NB), 1)
                < lax.broadcasted_iota(jnp.int32, (NB, NB), 0))
    cnt = jnp.zeros((NB, 1), jnp.float32)
    for ct in range(NBLK):
        sc_c = scr_ref[:, pl.ds(ct * NB, NB)]
        bat_c = batr_ref[:, pl.ds(ct * NB, NB)]
        tie_ok = (ct < i) | ((ct == i) & lj_lt_li)
        prec = (sc_c > sc_r) | ((sc_c == sc_r) & tie_ok)
        same = bat_c == bat_r
        cnt = cnt + jnp.sum((prec & same).astype(jnp.float32), axis=1,
                            keepdims=True)
    sel = (cnt < kofb_scr[pl.ds(r0, NB), :]) & (bat_r < G)

    gids = lax.broadcasted_iota(jnp.int32, (NB, G), 1)
    m = ((bat_r == gids) & sel).astype(jnp.float32)
    xwb = xw_ref[...]
    gap_scr[...] += lax.dot_general(m, xwb, (((0,), (0,)), ((), ())),
                                    preferred_element_type=jnp.float32)
    cnt_scr[...] += lax.dot_general(m, jnp.ones((NB, 128), jnp.float32),
                                    (((0,), (0,)), ((), ())),
                                    preferred_element_type=jnp.float32)
    for g in range(G):
        colm = m[:, g:g + 1] > 0.0
        vals = jnp.where(colm, xwb, -1e30)
        mx = jnp.max(vals, axis=0, keepdims=True)
        gmp_scr[g:g + 1, :] = jnp.maximum(gmp_scr[g:g + 1, :], mx)

    @pl.when(i == NBLK - 1)
    def _final():
        selc = jnp.maximum(cnt_scr[:, 0:1], 1.0)
        x2 = jnp.concatenate([gmp_scr[...], gap_scr[...] / selc], axis=1)
        h1 = jnp.maximum(jnp.dot(x2, l1_ref[...],
                                 preferred_element_type=jnp.float32)
                         + b1_ref[...], 0.0)
        h2 = jnp.maximum(jnp.dot(h1, l2_ref[...],
                                 preferred_element_type=jnp.float32)
                         + b2_ref[...], 0.0)
        logits = jnp.dot(h2, l3_ref[...],
                         preferred_element_type=jnp.float32) + b3_ref[...]
        mx = jnp.max(logits, axis=1, keepdims=True)
        lse = jnp.log(jnp.sum(jnp.exp(logits - mx), axis=1, keepdims=True))
        o_ref[...] = logits - mx - lse


def _tcb2(xw, scc, scr, batc, batr, l1, b1, l2p, b2p, l3p, b3p):
    return pl.pallas_call(
        _tcb2_body,
        grid=(NBLK,),
        in_specs=[
            pl.BlockSpec((NB, H * C), lambda i: (i, 0)),
            pl.BlockSpec((NP, 1), lambda i: (0, 0)),
            pl.BlockSpec((1, NP), lambda i: (0, 0)),
            pl.BlockSpec((NP, 1), lambda i: (0, 0)),
            pl.BlockSpec((1, NP), lambda i: (0, 0)),
            pl.BlockSpec((2 * H * C, 128), lambda i: (0, 0)),
            pl.BlockSpec((1, 128), lambda i: (0, 0)),
            pl.BlockSpec((128, 128), lambda i: (0, 0)),
            pl.BlockSpec((1, 128), lambda i: (0, 0)),
            pl.BlockSpec((128, 128), lambda i: (0, 0)),
            pl.BlockSpec((1, 128), lambda i: (0, 0)),
        ],
        out_specs=pl.BlockSpec((G, 128), lambda i: (0, 0)),
        out_shape=jax.ShapeDtypeStruct((G, 128), jnp.float32),
        scratch_shapes=[
            pltpu.VMEM((G, H * C), jnp.float32),
            pltpu.VMEM((G, H * C), jnp.float32),
            pltpu.VMEM((G, 128), jnp.float32),
            pltpu.VMEM((NP, 1), jnp.float32),
        ],
    )(xw, scc, scr, batc, batr, l1, b1, l2p, b2p, l3p, b3p)


def kernel(x, edge_index, batch, edge_attr, W_gat, att_src, att_dst, bias_gat,
           pool_w, lin1_W, lin1_b, lin2_W, lin2_b, lin3_W, lin3_b):
    f32 = jnp.float32
    xp = jnp.concatenate([x, jnp.zeros((NP - N, 32), f32)], axis=0)
    ones = jnp.concatenate([jnp.ones((N, 1), f32), jnp.zeros((NP - N, 1), f32)],
                           axis=0)

    a = pl.pallas_call(
        _tca_body,
        out_shape=jax.ShapeDtypeStruct((NP, 128), jnp.float32),
    )(xp, W_gat, att_src, att_dst)

    x_aug = jnp.concatenate(
        [xp, ones, a[:, 0:H], jnp.zeros((NP, RW - 37), f32)], axis=1)
    xd = jnp.concatenate(
        [jnp.zeros((NP, 1), f32), a[:, H:2 * H], jnp.zeros((NP, 11), f32)],
        axis=1)
    pad_e = jnp.full((EP - E + CH,), N, jnp.int32)
    srcp = jnp.concatenate([edge_index[0], pad_e]).reshape(EP // CH + 1, CH)
    dstp = jnp.concatenate([edge_index[1], pad_e]).reshape(EP // CH + 1, CH)

    u2f, u2d = _sc_edge(x_aug, srcp, dstp, xd)

    bias2 = bias_gat.reshape(1, H * C)
    pw2 = pool_w.reshape(1, H * C)
    xw, score = _tcb1(u2f, u2d, x_aug, a, W_gat, bias2, pw2)

    batp = jnp.concatenate([batch, jnp.full((NP - N,), G, jnp.int32)])
    batc = batp.reshape(NP, 1)
    batr = batp.reshape(1, NP)
    scc = score
    scr = score.reshape(1, NP)

    l2p = jnp.zeros((128, 128), f32).at[:, :64].set(lin2_W)
    b2p = jnp.zeros((1, 128), f32).at[0, :64].set(lin2_b)
    l3p = jnp.zeros((128, 128), f32).at[:64, :8].set(lin3_W)
    b3p = jnp.full((1, 128), -1e30, f32).at[0, :8].set(lin3_b)
    b1p = lin1_b.reshape(1, 128)

    out = _tcb2(xw, scc, scr, batc, batr, lin1_W, b1p, l2p, b2p, l3p, b3p)
    return out[:, :8]
